# baseline (speedup 1.0000x reference)
import functools

import jax
import jax.numpy as jnp
from jax import lax
from jax.experimental import pallas as pl
from jax.experimental.pallas import tpu as pltpu
from jax.experimental.pallas import tpu_sc as plsc

SEQ = 16384
EMBED = 64
NC = 2
NS = 16
NW = NC * NS
BPW = SEQ // NW
CHUNK = 128
NCH = BPW // CHUNK
LANES = 16
VECS_PER_ROW = EMBED // LANES


def _emb_body(x_hbm, tok_hbm, pos_hbm, out_hbm, idx_v, rows_v, pos_v, sem):
    wid = lax.axis_index("s") * NC + lax.axis_index("c")
    base = wid * BPW

    pltpu.sync_copy(x_hbm.at[wid], idx_v)

    gathers = [
        pltpu.make_async_copy(
            tok_hbm.at[idx_v.at[j]],
            rows_v.at[pl.ds(j * CHUNK, CHUNK)],
            sem,
        )
        for j in range(NCH)
    ]
    for g in gathers:
        g.start()
    pltpu.sync_copy(pos_hbm.at[pl.ds(base, BPW)], pos_v)
    for g in gathers:
        g.wait()

    def add_row(r, carry):
        for j in range(VECS_PER_ROW):
            sl = pl.ds(j * LANES, LANES)
            rows_v[r, sl] = rows_v[r, sl] + pos_v[r, sl]
        return carry

    lax.fori_loop(0, BPW, add_row, 0, unroll=4)

    pltpu.sync_copy(rows_v, out_hbm.at[pl.ds(base, BPW)])


def kernel(x, token_table, pos_table):
    x_r = x.astype(jnp.int32).reshape(NW, NCH, CHUNK)
    mesh = plsc.VectorSubcoreMesh(core_axis_name="c", subcore_axis_name="s")
    out = pl.kernel(
        _emb_body,
        mesh=mesh,
        out_type=jax.ShapeDtypeStruct((SEQ, EMBED), jnp.float32),
        scratch_types=[
            pltpu.VMEM((NCH, CHUNK), jnp.int32),
            pltpu.VMEM((BPW, EMBED), jnp.float32),
            pltpu.VMEM((BPW, EMBED), jnp.float32),
            pltpu.SemaphoreType.DMA,
        ],
        compiler_params=pltpu.CompilerParams(use_tc_tiling_on_sc=False),
    )(x_r, token_table, pos_table)
    return out[None]


# speedup vs baseline: 1.6768x; 1.6768x over previous
import functools

import jax
import jax.numpy as jnp
from jax import lax
from jax.experimental import pallas as pl
from jax.experimental.pallas import tpu as pltpu
from jax.experimental.pallas import tpu_sc as plsc

SEQ = 16384
EMBED = 64
NC = 2
NS = 16
NW = NC * NS
BPW = SEQ // NW
HS = 256
LANES = 16
VECS_PER_ROW = EMBED // LANES


def _emb_body(xr_hbm, tok_hbm, pos_hbm, out_hbm, idx_v, rows_v, pos_v, sem):
    wid = lax.axis_index("s") * NC + lax.axis_index("c")
    base = wid * BPW

    pltpu.sync_copy(xr_hbm.at[wid], idx_v)

    for h in range(BPW // HS):
        hbase = base + h * HS

        def fire(g, carry):
            v = idx_v[pl.ds(h * HS + g * LANES, LANES)]
            for l in range(LANES):
                pltpu.async_copy(tok_hbm.at[v[l]], rows_v.at[g * LANES + l], sem)
            return carry

        lax.fori_loop(0, HS // LANES, fire, 0)

        pltpu.sync_copy(pos_hbm.at[pl.ds(hbase, HS)], pos_v)

        pltpu.make_async_copy(tok_hbm.at[pl.ds(0, HS)], rows_v, sem).wait()

        def add_row(r, carry):
            for j in range(VECS_PER_ROW):
                sl = pl.ds(j * LANES, LANES)
                rows_v[r, sl] = rows_v[r, sl] + pos_v[r, sl]
            return carry

        lax.fori_loop(0, HS, add_row, 0, unroll=4)

        pltpu.sync_copy(rows_v, out_hbm.at[pl.ds(hbase, HS)])


def kernel(x, token_table, pos_table):
    xr = x.astype(jnp.int32).reshape(NW, BPW)
    mesh = plsc.VectorSubcoreMesh(core_axis_name="c", subcore_axis_name="s")
    out = pl.kernel(
        _emb_body,
        mesh=mesh,
        out_type=jax.ShapeDtypeStruct((SEQ, EMBED), jnp.float32),
        scratch_types=[
            pltpu.VMEM((BPW,), jnp.int32),
            pltpu.VMEM((HS, EMBED), jnp.float32),
            pltpu.VMEM((HS, EMBED), jnp.float32),
            pltpu.SemaphoreType.DMA,
        ],
    )(xr, token_table, pos_table)
    return out[None]
